# baseline (speedup 1.0000x reference)
import functools

import numpy as np
import jax
import jax.numpy as jnp
from jax import lax
from jax.experimental import pallas as pl
from jax.experimental.pallas import tpu as pltpu
from jax.experimental.pallas import tpu_sc as plsc

D_MODEL = 768
VOCAB = 100000
BATCH = 4
SEQ = 8192

SCALE = float(np.sqrt(np.float32(D_MODEL)))

NUM_CORES = 2
NUM_SUBCORES = 16
NUM_WORKERS = NUM_CORES * NUM_SUBCORES
POS_PER_WORKER = SEQ // NUM_WORKERS
SUPER = 16
N_STEPS = POS_PER_WORKER // SUPER
LANES = 16
D_GROUPS = D_MODEL // LANES
PE_BLOCK = D_GROUPS // 8


def _sinusoidal_pe(length, d_model):
    pos = np.arange(length)[:, None].astype(np.float32)
    i = np.arange(d_model)[None, :].astype(np.float32)
    angle_rates = 1.0 / np.power(10000.0, (2.0 * (i // 2)) / np.float32(d_model))
    angles = pos * angle_rates
    pe = np.zeros((length, d_model), dtype=np.float32)
    pe[:, 0::2] = np.sin(angles[:, 0::2])
    pe[:, 1::2] = np.cos(angles[:, 1::2])
    return pe


_PE = _sinusoidal_pe(SEQ, D_MODEL)

_MESH = plsc.VectorSubcoreMesh(core_axis_name="c", subcore_axis_name="s")

_BUF = pltpu.VMEM((SUPER, D_MODEL), jnp.float32)


@functools.partial(
    pl.kernel,
    out_type=jax.ShapeDtypeStruct((BATCH, SEQ, D_MODEL), jnp.float32),
    mesh=_MESH,
    scratch_types=[
        pltpu.VMEM((BATCH, POS_PER_WORKER), jnp.int32),
        _BUF, _BUF, _BUF, _BUF,
        _BUF, _BUF, _BUF, _BUF,
        _BUF, _BUF,
        pltpu.SemaphoreType.DMA, pltpu.SemaphoreType.DMA,
        pltpu.SemaphoreType.DMA, pltpu.SemaphoreType.DMA,
        pltpu.SemaphoreType.DMA, pltpu.SemaphoreType.DMA,
        pltpu.SemaphoreType.DMA, pltpu.SemaphoreType.DMA,
        pltpu.SemaphoreType.DMA, pltpu.SemaphoreType.DMA,
    ],
)
def _emb_pe_kernel(x_hbm, table_hbm, pe_hbm, out_hbm,
                   idx_v,
                   ga0, ga1, ga2, ga3, gb0, gb1, gb2, gb3,
                   pe0, pe1,
                   sa0, sa1, sa2, sa3, sb0, sb1, sb2, sb3,
                   pf0, pf1):
    gbuf = ((ga0, ga1, ga2, ga3), (gb0, gb1, gb2, gb3))
    pebuf = (pe0, pe1)
    gsem = ((sa0, sa1, sa2, sa3), (sb0, sb1, sb2, sb3))
    fsem = (pf0, pf1)

    wid = lax.axis_index("s") * NUM_CORES + lax.axis_index("c")
    pos0 = wid * POS_PER_WORKER

    def pe_src(t):
        return pe_hbm.at[pl.ds(pos0 + t * SUPER, SUPER)]

    def gather_src(t, b):
        return table_hbm.at[idx_v.at[b, pl.ds(t * SUPER, SUPER)]]

    def out_dst(t, b):
        return out_hbm.at[b, pl.ds(pos0 + t * SUPER, SUPER)]

    def issue_f(t, q):
        pltpu.make_async_copy(pe_src(t), pebuf[q], fsem[q]).start()

    def wait_f(t, q):
        pltpu.make_async_copy(pe_src(t), pebuf[q], fsem[q]).wait()

    def issue_g(t, q, b):
        pltpu.make_async_copy(gather_src(t, b), gbuf[q][b], gsem[q][b]).start()

    def wait_g(t, q, b):
        pltpu.make_async_copy(gather_src(t, b), gbuf[q][b], gsem[q][b]).wait()

    def issue_s(t, q, b):
        pltpu.make_async_copy(gbuf[q][b], out_dst(t, b), gsem[q][b]).start()

    def wait_s(t, q, b):
        pltpu.make_async_copy(gbuf[q][b], out_dst(t, b), gsem[q][b]).wait()

    def compute(q):
        pe_v = pebuf[q]
        rows = gbuf[q]

        @plsc.parallel_loop(0, SUPER)
        def _(r):
            @plsc.parallel_loop(0, D_MODEL, step=LANES, unroll=8)
            def _(c):
                sl = pl.ds(c, LANES)
                p = pe_v[r, sl]
                for b in range(BATCH):
                    g = rows[b]
                    g[r, sl] = g[r, sl] * SCALE + p

    for b in range(BATCH):
        pltpu.sync_copy(x_hbm.at[b, pl.ds(pos0, POS_PER_WORKER)],
                        idx_v.at[b])

    issue_f(0, 0)
    for b in range(BATCH):
        issue_g(0, 0, b)

    def sub_step(t, q, j):
        wait_f(t, q)

        @pl.when(t < N_STEPS - 1)
        def _():
            issue_f(t + 1, 1 - q)
        for b in range(BATCH):
            wait_g(t, q, b)
        @pl.when(t > 0)
        def _():
            for b in range(BATCH):
                wait_s(t - 1, 1 - q, b)

        @pl.when(t < N_STEPS - 1)
        def _():
            for b in range(BATCH):
                issue_g(t + 1, 1 - q, b)
        compute(q)
        for b in range(BATCH):
            issue_s(t, q, b)

    def pair_body(j, carry):
        sub_step(2 * j, 0, j)
        sub_step(2 * j + 1, 1, j)
        return carry

    lax.fori_loop(0, N_STEPS // 2, pair_body, 0)

    for b in range(BATCH):
        wait_s(N_STEPS - 1, (N_STEPS - 1) % 2, b)


def kernel(x, table):
    pe = jnp.asarray(_PE)
    return _emb_pe_kernel(x.astype(jnp.int32), table, pe)


# speedup vs baseline: 1.0051x; 1.0051x over previous
import functools

import numpy as np
import jax
import jax.numpy as jnp
from jax import lax
from jax.experimental import pallas as pl
from jax.experimental.pallas import tpu as pltpu
from jax.experimental.pallas import tpu_sc as plsc

D_MODEL = 768
VOCAB = 100000
BATCH = 4
SEQ = 8192

SCALE = float(np.sqrt(np.float32(D_MODEL)))

NUM_CORES = 2
NUM_SUBCORES = 16
NUM_WORKERS = NUM_CORES * NUM_SUBCORES
POS_PER_WORKER = SEQ // NUM_WORKERS
SUPER = 16
N_STEPS = POS_PER_WORKER // SUPER
LANES = 16
D_GROUPS = D_MODEL // LANES


def _sinusoidal_pe(length, d_model):
    pos = np.arange(length)[:, None].astype(np.float32)
    i = np.arange(d_model)[None, :].astype(np.float32)
    angle_rates = 1.0 / np.power(10000.0, (2.0 * (i // 2)) / np.float32(d_model))
    angles = pos * angle_rates
    pe = np.zeros((length, d_model), dtype=np.float32)
    pe[:, 0::2] = np.sin(angles[:, 0::2])
    pe[:, 1::2] = np.cos(angles[:, 1::2])
    return pe


_PE = _sinusoidal_pe(SEQ, D_MODEL)

_MESH = plsc.VectorSubcoreMesh(core_axis_name="c", subcore_axis_name="s")

_BUF = pltpu.VMEM((SUPER, D_MODEL), jnp.float32)


@functools.partial(
    pl.kernel,
    out_type=jax.ShapeDtypeStruct((BATCH, SEQ, D_MODEL), jnp.float32),
    mesh=_MESH,
    scratch_types=[
        pltpu.VMEM((BATCH, POS_PER_WORKER), jnp.int32),
        _BUF, _BUF, _BUF, _BUF,
        _BUF, _BUF, _BUF, _BUF,
        _BUF, _BUF,
        pltpu.SemaphoreType.DMA, pltpu.SemaphoreType.DMA,
        pltpu.SemaphoreType.DMA, pltpu.SemaphoreType.DMA,
        pltpu.SemaphoreType.DMA, pltpu.SemaphoreType.DMA,
        pltpu.SemaphoreType.DMA, pltpu.SemaphoreType.DMA,
        pltpu.SemaphoreType.DMA, pltpu.SemaphoreType.DMA,
    ],
)
def _emb_pe_kernel(x_hbm, table_hbm, pe_hbm, out_hbm,
                   idx_v,
                   ga0, ga1, ga2, ga3, gb0, gb1, gb2, gb3,
                   pe0, pe1,
                   sa0, sa1, sa2, sa3, sb0, sb1, sb2, sb3,
                   pf0, pf1):
    gbuf = ((ga0, ga1, ga2, ga3), (gb0, gb1, gb2, gb3))
    pebuf = (pe0, pe1)
    gsem = ((sa0, sa1, sa2, sa3), (sb0, sb1, sb2, sb3))
    fsem = (pf0, pf1)

    wid = lax.axis_index("s") * NUM_CORES + lax.axis_index("c")
    pos0 = wid * POS_PER_WORKER

    def pe_src(t):
        return pe_hbm.at[pl.ds(pos0 + t * SUPER, SUPER)]

    def gather_src(t, b):
        return table_hbm.at[idx_v.at[b, pl.ds(t * SUPER, SUPER)]]

    def out_dst(t, b):
        return out_hbm.at[b, pl.ds(pos0 + t * SUPER, SUPER)]

    def issue_f(t, q):
        pltpu.make_async_copy(pe_src(t), pebuf[q], fsem[q]).start()

    def wait_f(t, q):
        pltpu.make_async_copy(pe_src(t), pebuf[q], fsem[q]).wait()

    def issue_g(t, q, b):
        pltpu.make_async_copy(gather_src(t, b), gbuf[q][b], gsem[q][b]).start()

    def wait_g(t, q, b):
        pltpu.make_async_copy(gather_src(t, b), gbuf[q][b], gsem[q][b]).wait()

    def issue_s(t, q, b):
        pltpu.make_async_copy(gbuf[q][b], out_dst(t, b), gsem[q][b]).start()

    def wait_s(t, q, b):
        pltpu.make_async_copy(gbuf[q][b], out_dst(t, b), gsem[q][b]).wait()

    def compute(q):
        pe_v = pebuf[q]
        rows = gbuf[q]

        @plsc.parallel_loop(0, SUPER)
        def _(r):
            @plsc.parallel_loop(0, D_MODEL, step=LANES, unroll=4)
            def _(c):
                sl = pl.ds(c, LANES)
                p = pe_v[r, sl]
                for b in range(BATCH):
                    g = rows[b]
                    g[r, sl] = g[r, sl] * SCALE + p

    for b in range(BATCH):
        pltpu.sync_copy(x_hbm.at[b, pl.ds(pos0, POS_PER_WORKER)],
                        idx_v.at[b])

    issue_f(0, 0)
    for b in range(BATCH):
        issue_g(0, 0, b)

    def sub_step(t, q, j):
        wait_f(t, q)

        @pl.when(t < N_STEPS - 1)
        def _():
            issue_f(t + 1, 1 - q)
        for b in range(BATCH):
            wait_g(t, q, b)
        @pl.when(t > 0)
        def _():
            for b in range(BATCH):
                wait_s(t - 1, 1 - q, b)

        @pl.when(t < N_STEPS - 1)
        def _():
            for b in range(BATCH):
                issue_g(t + 1, 1 - q, b)
        compute(q)
        for b in range(BATCH):
            issue_s(t, q, b)

    def pair_body(j, carry):
        sub_step(2 * j, 0, j)
        sub_step(2 * j + 1, 1, j)
        return carry

    lax.fori_loop(0, N_STEPS // 2, pair_body, 0)

    for b in range(BATCH):
        wait_s(N_STEPS - 1, (N_STEPS - 1) % 2, b)


def kernel(x, table):
    pe = jnp.asarray(_PE)
    return _emb_pe_kernel(x.astype(jnp.int32), table, pe)


# speedup vs baseline: 1.0165x; 1.0113x over previous
import functools

import numpy as np
import jax
import jax.numpy as jnp
from jax import lax
from jax.experimental import pallas as pl
from jax.experimental.pallas import tpu as pltpu
from jax.experimental.pallas import tpu_sc as plsc

D_MODEL = 768
VOCAB = 100000
BATCH = 4
SEQ = 8192

SCALE = float(np.sqrt(np.float32(D_MODEL)))

NUM_CORES = 2
NUM_SUBCORES = 16
NUM_WORKERS = NUM_CORES * NUM_SUBCORES
POS_PER_WORKER = SEQ // NUM_WORKERS
SUPER = 16
N_STEPS = POS_PER_WORKER // SUPER
LANES = 16
D_GROUPS = D_MODEL // LANES


def _sinusoidal_pe(length, d_model):
    pos = np.arange(length)[:, None].astype(np.float32)
    i = np.arange(d_model)[None, :].astype(np.float32)
    angle_rates = 1.0 / np.power(10000.0, (2.0 * (i // 2)) / np.float32(d_model))
    angles = pos * angle_rates
    pe = np.zeros((length, d_model), dtype=np.float32)
    pe[:, 0::2] = np.sin(angles[:, 0::2])
    pe[:, 1::2] = np.cos(angles[:, 1::2])
    return pe


_PE = _sinusoidal_pe(SEQ, D_MODEL)

_MESH = plsc.VectorSubcoreMesh(core_axis_name="c", subcore_axis_name="s")

_BUF = pltpu.VMEM((SUPER, D_MODEL), jnp.float32)


@functools.partial(
    pl.kernel,
    out_type=jax.ShapeDtypeStruct((BATCH, SEQ, D_MODEL), jnp.float32),
    mesh=_MESH,
    scratch_types=[
        pltpu.VMEM((BATCH, POS_PER_WORKER), jnp.int32),
        _BUF, _BUF, _BUF, _BUF,
        _BUF, _BUF, _BUF, _BUF,
        _BUF, _BUF,
        pltpu.SemaphoreType.DMA, pltpu.SemaphoreType.DMA,
        pltpu.SemaphoreType.DMA, pltpu.SemaphoreType.DMA,
        pltpu.SemaphoreType.DMA, pltpu.SemaphoreType.DMA,
        pltpu.SemaphoreType.DMA, pltpu.SemaphoreType.DMA,
        pltpu.SemaphoreType.DMA, pltpu.SemaphoreType.DMA,
    ],
)
def _emb_pe_kernel(x_hbm, table_hbm, pe_hbm, out_hbm,
                   idx_v,
                   ga0, ga1, ga2, ga3, gb0, gb1, gb2, gb3,
                   pe0, pe1,
                   sa0, sa1, sa2, sa3, sb0, sb1, sb2, sb3,
                   pf0, pf1):
    gbuf = ((ga0, ga1, ga2, ga3), (gb0, gb1, gb2, gb3))
    pebuf = (pe0, pe1)
    gsem = ((sa0, sa1, sa2, sa3), (sb0, sb1, sb2, sb3))
    fsem = (pf0, pf1)

    wid = lax.axis_index("s") * NUM_CORES + lax.axis_index("c")
    pos0 = wid * POS_PER_WORKER

    def pe_src(t):
        return pe_hbm.at[pl.ds(pos0 + t * SUPER, SUPER)]

    def gather_src(t, b):
        return table_hbm.at[idx_v.at[b, pl.ds(t * SUPER, SUPER)]]

    def out_dst(t, b):
        return out_hbm.at[b, pl.ds(pos0 + t * SUPER, SUPER)]

    def issue_f(t, q):
        pltpu.make_async_copy(pe_src(t), pebuf[q], fsem[q]).start()

    def wait_f(t, q):
        pltpu.make_async_copy(pe_src(t), pebuf[q], fsem[q]).wait()

    def issue_g(t, q, b):
        pltpu.make_async_copy(gather_src(t, b), gbuf[q][b], gsem[q][b]).start()

    def wait_g(t, q, b):
        pltpu.make_async_copy(gather_src(t, b), gbuf[q][b], gsem[q][b]).wait()

    def issue_s(t, q, b):
        pltpu.make_async_copy(gbuf[q][b], out_dst(t, b), gsem[q][b]).start()

    def wait_s(t, q, b):
        pltpu.make_async_copy(gbuf[q][b], out_dst(t, b), gsem[q][b]).wait()

    def compute(q):
        pe_v = pebuf[q]
        rows = gbuf[q]

        @plsc.parallel_loop(0, SUPER)
        def _(r):
            @plsc.parallel_loop(0, D_MODEL, step=LANES, unroll=4)
            def _(c):
                sl = pl.ds(c, LANES)
                p = pe_v[r, sl]
                for b in range(BATCH):
                    g = rows[b]
                    g[r, sl] = g[r, sl] * SCALE + p

    for b in range(BATCH):
        pltpu.make_async_copy(x_hbm.at[b, pl.ds(pos0, POS_PER_WORKER)],
                              idx_v.at[b], gsem[0][b]).start()
    issue_f(0, 0)
    for b in range(BATCH):
        pltpu.make_async_copy(x_hbm.at[b, pl.ds(pos0, POS_PER_WORKER)],
                              idx_v.at[b], gsem[0][b]).wait()
    for b in range(BATCH):
        issue_g(0, 0, b)

    def sub_step(t, q, j):
        wait_f(t, q)

        @pl.when(t < N_STEPS - 1)
        def _():
            issue_f(t + 1, 1 - q)
        for b in range(BATCH):
            wait_g(t, q, b)
        @pl.when(t > 0)
        def _():
            for b in range(BATCH):
                wait_s(t - 1, 1 - q, b)

        @pl.when(t < N_STEPS - 1)
        def _():
            for b in range(BATCH):
                issue_g(t + 1, 1 - q, b)
        compute(q)
        for b in range(BATCH):
            issue_s(t, q, b)

    def pair_body(j, carry):
        sub_step(2 * j, 0, j)
        sub_step(2 * j + 1, 1, j)
        return carry

    lax.fori_loop(0, N_STEPS // 2, pair_body, 0)

    for b in range(BATCH):
        wait_s(N_STEPS - 1, (N_STEPS - 1) % 2, b)


def kernel(x, table):
    pe = jnp.asarray(_PE)
    return _emb_pe_kernel(x.astype(jnp.int32), table, pe)


# speedup vs baseline: 1.0181x; 1.0016x over previous
import functools

import numpy as np
import jax
import jax.numpy as jnp
from jax import lax
from jax.experimental import pallas as pl
from jax.experimental.pallas import tpu as pltpu
from jax.experimental.pallas import tpu_sc as plsc

D_MODEL = 768
VOCAB = 100000
BATCH = 4
SEQ = 8192

SCALE = float(np.sqrt(np.float32(D_MODEL)))

NUM_CORES = 2
NUM_SUBCORES = 16
NUM_WORKERS = NUM_CORES * NUM_SUBCORES
POS_PER_WORKER = SEQ // NUM_WORKERS
SUPER = 16
N_STEPS = POS_PER_WORKER // SUPER
LANES = 16
D_GROUPS = D_MODEL // LANES


def _sinusoidal_pe(length, d_model):
    pos = np.arange(length)[:, None].astype(np.float32)
    i = np.arange(d_model)[None, :].astype(np.float32)
    angle_rates = 1.0 / np.power(10000.0, (2.0 * (i // 2)) / np.float32(d_model))
    angles = pos * angle_rates
    pe = np.zeros((length, d_model), dtype=np.float32)
    pe[:, 0::2] = np.sin(angles[:, 0::2])
    pe[:, 1::2] = np.cos(angles[:, 1::2])
    return pe


_PE = _sinusoidal_pe(SEQ, D_MODEL)

_MESH = plsc.VectorSubcoreMesh(core_axis_name="c", subcore_axis_name="s")

_BUF = pltpu.VMEM((SUPER, D_MODEL), jnp.float32)


@functools.partial(
    pl.kernel,
    out_type=jax.ShapeDtypeStruct((BATCH, SEQ, D_MODEL), jnp.float32),
    mesh=_MESH,
    scratch_types=[
        pltpu.VMEM((BATCH, POS_PER_WORKER), jnp.int32),
        _BUF, _BUF, _BUF, _BUF,
        _BUF, _BUF, _BUF, _BUF,
        _BUF, _BUF,
        pltpu.SemaphoreType.DMA, pltpu.SemaphoreType.DMA,
        pltpu.SemaphoreType.DMA, pltpu.SemaphoreType.DMA,
        pltpu.SemaphoreType.DMA, pltpu.SemaphoreType.DMA,
        pltpu.SemaphoreType.DMA, pltpu.SemaphoreType.DMA,
        pltpu.SemaphoreType.DMA, pltpu.SemaphoreType.DMA,
    ],
)
def _emb_pe_kernel(x_hbm, table_hbm, pe_hbm, out_hbm,
                   idx_v,
                   ga0, ga1, ga2, ga3, gb0, gb1, gb2, gb3,
                   pe0, pe1,
                   sa0, sa1, sa2, sa3, sb0, sb1, sb2, sb3,
                   pf0, pf1):
    gbuf = ((ga0, ga1, ga2, ga3), (gb0, gb1, gb2, gb3))
    pebuf = (pe0, pe1)
    gsem = ((sa0, sa1, sa2, sa3), (sb0, sb1, sb2, sb3))
    fsem = (pf0, pf1)

    wid = lax.axis_index("s") * NUM_CORES + lax.axis_index("c")
    pos0 = wid * POS_PER_WORKER

    def pe_src(t):
        return pe_hbm.at[pl.ds(pos0 + t * SUPER, SUPER)]

    def gather_src(t, b):
        return table_hbm.at[idx_v.at[b, pl.ds(t * SUPER, SUPER)]]

    def out_dst(t, b):
        return out_hbm.at[b, pl.ds(pos0 + t * SUPER, SUPER)]

    def issue_f(t, q):
        pltpu.make_async_copy(pe_src(t), pebuf[q], fsem[q]).start()

    def wait_f(t, q):
        pltpu.make_async_copy(pe_src(t), pebuf[q], fsem[q]).wait()

    def issue_g(t, q, b):
        pltpu.make_async_copy(gather_src(t, b), gbuf[q][b], gsem[q][b]).start()

    def wait_g(t, q, b):
        pltpu.make_async_copy(gather_src(t, b), gbuf[q][b], gsem[q][b]).wait()

    def issue_s(t, q, b):
        pltpu.make_async_copy(gbuf[q][b], out_dst(t, b), gsem[q][b]).start()

    def wait_s(t, q, b):
        pltpu.make_async_copy(gbuf[q][b], out_dst(t, b), gsem[q][b]).wait()

    def compute(q):
        pe_v = pebuf[q]
        rows = gbuf[q]

        @plsc.parallel_loop(0, SUPER)
        def _(r):
            @plsc.parallel_loop(0, D_MODEL, step=LANES, unroll=4)
            def _(c):
                sl = pl.ds(c, LANES)
                p = pe_v[r, sl]
                for b in range(BATCH):
                    g = rows[b]
                    g[r, sl] = g[r, sl] * SCALE + p

    for b in range(BATCH):
        pltpu.make_async_copy(x_hbm.at[b, pl.ds(pos0, POS_PER_WORKER)],
                              idx_v.at[b], gsem[0][b]).start()
    issue_f(0, 0)
    for b in range(BATCH):
        pltpu.make_async_copy(x_hbm.at[b, pl.ds(pos0, POS_PER_WORKER)],
                              idx_v.at[b], gsem[0][b]).wait()
    for b in range(BATCH):
        issue_g(0, 0, b)

    def sub_step(t, q):
        wait_f(t, q)

        @pl.when(t < N_STEPS - 1)
        def _():
            issue_f(t + 1, 1 - q)
        for b in range(BATCH):
            wait_g(t, q, b)
        @pl.when(t > 0)
        def _():
            for b in range(BATCH):
                wait_s(t - 1, 1 - q, b)

        @pl.when(t < N_STEPS - 1)
        def _():
            for b in range(BATCH):
                issue_g(t + 1, 1 - q, b)
        compute(q)
        for b in range(BATCH):
            issue_s(t, q, b)

    def pair_body(j, carry):
        sub_step(2 * j, 0)
        sub_step(2 * j + 1, 1)
        return carry

    lax.fori_loop(0, N_STEPS // 2, pair_body, 0)

    for b in range(BATCH):
        wait_s(N_STEPS - 1, (N_STEPS - 1) % 2, b)


def kernel(x, table):
    pe = jnp.asarray(_PE)
    return _emb_pe_kernel(x.astype(jnp.int32), table, pe)
